# Initial kernel scaffold; baseline (speedup 1.0000x reference)
import jax
import jax.numpy as jnp
from jax.experimental import pallas as pl


def kernel(xt, rewards, subset, action):
    raise NotImplementedError("write your pallas kernel here")


# speedup vs baseline: 1.0374x; 1.0374x over previous
import jax
import jax.numpy as jnp
from jax.experimental import pallas as pl

B = 4096
DU = 768
DA = 256
HEADS = 8
BB = 512


def _body(xt_ref, rew_ref, sub_ref, act_ref, out_ref):
    m = (rew_ref[:, 0:1] == 1) & (sub_ref[0, :, :] >= 0.1)
    sel = jnp.where(m, xt_ref[...], 0.0)
    act = jnp.broadcast_to(act_ref[0, :, :], (BB, DA))
    out_ref[0, :, :] = jnp.concatenate([sel, act], axis=1)


def kernel(xt, rewards, subset, action):
    xt2 = xt.reshape(B, DU)
    rew2 = rewards.reshape(B, 1)
    sub3 = subset.T.reshape(HEADS, B, 1)
    act3 = action.reshape(HEADS, 1, DA)
    out = pl.pallas_call(
        _body,
        grid=(B // BB, HEADS),
        in_specs=[
            pl.BlockSpec((BB, DU), lambda i, h: (i, 0)),
            pl.BlockSpec((BB, 1), lambda i, h: (i, 0)),
            pl.BlockSpec((1, BB, 1), lambda i, h: (h, i, 0)),
            pl.BlockSpec((1, 1, DA), lambda i, h: (h, 0, 0)),
        ],
        out_specs=pl.BlockSpec((1, BB, DU + DA), lambda i, h: (h, i, 0)),
        out_shape=jax.ShapeDtypeStruct((HEADS, B, DU + DA), jnp.float32),
    )(xt2, rew2, sub3, act3)
    return out.reshape(HEADS, B, 1, DU + DA)
